# baseline (speedup 1.0000x reference)
import functools

import jax
import jax.numpy as jnp
from jax.experimental import pallas as pl
from jax.experimental.pallas import tpu as pltpu

K = 64


def _encode_body(x_ref, w_ref, benc_ref, bdec_ref, f_ref):
    xb = x_ref[...] - bdec_ref[...]
    acc = jax.lax.dot_general(
        xb, w_ref[...],
        dimension_numbers=(((1,), (1,)), ((), ())),
        preferred_element_type=jnp.float32,
        precision=jax.lax.Precision.DEFAULT,
    )
    f_ref[...] = jnp.maximum(acc + benc_ref[...], 0.0)


def _encode(x, W_enc, b_enc, b_dec, tm, tn):
    n, d = x.shape
    f_dim = W_enc.shape[0]
    grid = (f_dim // tn, n // tm)
    return pl.pallas_call(
        _encode_body,
        grid=grid,
        in_specs=[
            pl.BlockSpec((tm, d), lambda j, i: (i, 0)),
            pl.BlockSpec((tn, d), lambda j, i: (j, 0)),
            pl.BlockSpec((1, tn), lambda j, i: (0, j)),
            pl.BlockSpec((1, d), lambda j, i: (0, 0)),
        ],
        out_specs=pl.BlockSpec((tm, tn), lambda j, i: (i, j)),
        out_shape=jax.ShapeDtypeStruct((n, f_dim), jnp.float32),
    )(x, W_enc, b_enc.reshape(1, f_dim), b_dec.reshape(1, d))


def _count_ge_s16(arr_ref, cand, acc_w=512):
    tm, w = arr_ref.shape
    cand16 = cand.astype(jnp.int16)
    acc = jnp.zeros((tm, acc_w), jnp.int16)
    for i in range(w // acc_w):
        chunk = arr_ref[:, i * acc_w : (i + 1) * acc_w]
        m = chunk >= cand16
        acc = acc + jnp.where(m, jnp.int16(1), jnp.int16(0))
    return jnp.sum(acc.astype(jnp.int32), axis=1, keepdims=True)


def _threshold_body(f_ref, t_ref, hz_ref):
    tm, f_dim = f_ref.shape
    n_ch = 32
    c_ch = f_dim // n_ch

    for i in range(n_ch):
        sl = slice(i * c_ch, (i + 1) * c_ch)
        fi0 = jax.lax.bitcast_convert_type(f_ref[:, sl], jnp.int32)
        hz_ref[:, sl] = jax.lax.shift_right_logical(fi0, 16).astype(jnp.int16)

    def body_hi(b, t):
        bit = jnp.int32(1) << (jnp.int32(14) - b)
        cand = t | bit
        cnt = _count_ge_s16(hz_ref, cand)
        return jnp.where(cnt >= K, cand, t)

    t16 = jax.lax.fori_loop(0, 15, body_hi, jnp.zeros((tm, 1), jnp.int32))
    c_gt = _count_ge_s16(hz_ref, t16 + 1)

    t16_16 = t16.astype(jnp.int16)
    for i in range(n_ch):
        sl = slice(i * c_ch, (i + 1) * c_ch)
        m_eq = hz_ref[:, sl] == t16_16
        fi1 = jax.lax.bitcast_convert_type(f_ref[:, sl], jnp.int32)
        lo15 = (jax.lax.shift_right_logical(fi1, 1)
                & jnp.int32(0x7FFF)).astype(jnp.int16)
        hz_ref[:, sl] = jnp.where(m_eq, lo15, jnp.int16(-1))

    def body_lo(b, q):
        bit = jnp.int32(1) << (jnp.int32(14) - b)
        cand = q | bit
        cnt = c_gt + _count_ge_s16(hz_ref, cand)
        return jnp.where(cnt >= K, cand, q)

    q = jax.lax.fori_loop(0, 15, body_lo, jnp.zeros((tm, 1), jnp.int32))

    cand_pat = (t16 << 16) | (q << 1) | 1
    cand_f = jax.lax.bitcast_convert_type(cand_pat, jnp.float32)
    n_sl = 8
    c_sl = f_dim // n_sl
    parts = [
        jnp.sum(
            (f_ref[:, i * c_sl : (i + 1) * c_sl] >= cand_f).astype(jnp.float32),
            axis=1, keepdims=True)
        for i in range(n_sl)
    ]
    cnt_full = functools.reduce(jnp.add, parts).astype(jnp.int32)
    t_pat = jnp.where(cnt_full >= K, cand_pat, cand_pat ^ 1)
    t_ref[...] = jax.lax.bitcast_convert_type(t_pat, jnp.float32)


def _threshold(f, tm):
    n, f_dim = f.shape
    return pl.pallas_call(
        _threshold_body,
        grid=(n // tm,),
        in_specs=[pl.BlockSpec((tm, f_dim), lambda i: (i, 0))],
        out_specs=pl.BlockSpec((tm, 1), lambda i: (i, 0)),
        out_shape=jax.ShapeDtypeStruct((n, 1), jnp.float32),
        scratch_shapes=[pltpu.VMEM((tm, f_dim), jnp.int16)],
    )(f)


def _decode_body(f_ref, t_ref, dec_ref, bdec_ref, o_ref, vm_ref):
    j = pl.program_id(1)
    tf_ = t_ref[...]
    tf_dim = f_ref.shape[1]
    n_ch = 4
    c_ch = tf_dim // n_ch
    for i in range(n_ch):
        sl = slice(i * c_ch, (i + 1) * c_ch)
        fb = f_ref[:, sl]
        vm_ref[:, sl] = jnp.where(
            (fb >= tf_) & (fb > 0.0), fb, 0.0).astype(jnp.bfloat16)
    part = jax.lax.dot_general(
        vm_ref[...], dec_ref[...],
        dimension_numbers=(((1,), (0,)), ((), ())),
        preferred_element_type=jnp.float32,
    )

    @pl.when(j == 0)
    def _():
        o_ref[...] = part + bdec_ref[...]

    @pl.when(j > 0)
    def _():
        o_ref[...] += part


def _decode(f, t, decoder, b_dec, tm, tf):
    n, f_dim = f.shape
    d = decoder.shape[1]
    grid = (n // tm, f_dim // tf)
    return pl.pallas_call(
        _decode_body,
        grid=grid,
        in_specs=[
            pl.BlockSpec((tm, tf), lambda i, j: (i, j)),
            pl.BlockSpec((tm, 1), lambda i, j: (i, 0)),
            pl.BlockSpec((tf, d), lambda i, j: (j, 0)),
            pl.BlockSpec((1, d), lambda i, j: (0, 0)),
        ],
        out_specs=pl.BlockSpec((tm, d), lambda i, j: (i, 0)),
        out_shape=jax.ShapeDtypeStruct((n, d), jnp.float32),
        scratch_shapes=[pltpu.VMEM((tm, tf), jnp.bfloat16)],
    )(f, t, decoder, b_dec.reshape(1, d))


@functools.partial(jax.jit, static_argnames=())
def kernel(x, W_enc, b_enc, decoder, b_dec):
    orig_shape = x.shape
    x2 = x.reshape(-1, orig_shape[-1])
    n, d = x2.shape
    f_dim = W_enc.shape[0]

    tm_e = min(512, n)
    tn_e = min(1024, f_dim)
    f = _encode(x2, W_enc, b_enc, b_dec, tm_e, tn_e)

    tm_t = min(256, n)
    t = _threshold(f, tm_t)

    tm_d = min(1024, n)
    tf_d = min(1024, f_dim)
    x_hat = _decode(f, t, decoder.astype(jnp.bfloat16), b_dec, tm_d, tf_d)
    return x_hat.reshape(orig_shape)


# speedup vs baseline: 1.0166x; 1.0166x over previous
import functools

import jax
import jax.numpy as jnp
from jax.experimental import pallas as pl
from jax.experimental.pallas import tpu as pltpu

K = 64


def _encode_body(x_ref, w_ref, benc_ref, bdec_ref, f_ref):
    xb = x_ref[...] - bdec_ref[...]
    acc = jax.lax.dot_general(
        xb, w_ref[...],
        dimension_numbers=(((1,), (1,)), ((), ())),
        preferred_element_type=jnp.float32,
        precision=jax.lax.Precision.DEFAULT,
    )
    f_ref[...] = jnp.maximum(acc + benc_ref[...], 0.0)


def _encode(x, W_enc, b_enc, b_dec, tm, tn):
    n, d = x.shape
    f_dim = W_enc.shape[0]
    grid = (f_dim // tn, n // tm)
    return pl.pallas_call(
        _encode_body,
        grid=grid,
        in_specs=[
            pl.BlockSpec((tm, d), lambda j, i: (i, 0)),
            pl.BlockSpec((tn, d), lambda j, i: (j, 0)),
            pl.BlockSpec((1, tn), lambda j, i: (0, j)),
            pl.BlockSpec((1, d), lambda j, i: (0, 0)),
        ],
        out_specs=pl.BlockSpec((tm, tn), lambda j, i: (i, j)),
        out_shape=jax.ShapeDtypeStruct((n, f_dim), jnp.float32),
    )(x, W_enc, b_enc.reshape(1, f_dim), b_dec.reshape(1, d))


def _count_ge_s16(arr_ref, cand, acc_w=512):
    tm, w = arr_ref.shape
    cand16 = cand.astype(jnp.int16)
    acc = jnp.zeros((tm, acc_w), jnp.int16)
    for i in range(w // acc_w):
        chunk = arr_ref[:, i * acc_w : (i + 1) * acc_w]
        m = chunk >= cand16
        acc = acc + jnp.where(m, jnp.int16(1), jnp.int16(0))
    return jnp.sum(acc.astype(jnp.int32), axis=1, keepdims=True)


def _threshold_body(f_ref, t_ref, hz_ref):
    tm, f_dim = f_ref.shape
    n_ch = 32
    c_ch = f_dim // n_ch

    for i in range(n_ch):
        sl = slice(i * c_ch, (i + 1) * c_ch)
        fi0 = jax.lax.bitcast_convert_type(f_ref[:, sl], jnp.int32)
        hz_ref[:, sl] = jax.lax.shift_right_logical(fi0, 16).astype(jnp.int16)

    def body_hi(b, t):
        bit = jnp.int32(1) << (jnp.int32(14) - b)
        cand = t | bit
        cnt = _count_ge_s16(hz_ref, cand)
        return jnp.where(cnt >= K, cand, t)

    t16 = jax.lax.fori_loop(0, 15, body_hi, jnp.zeros((tm, 1), jnp.int32))
    c_gt = _count_ge_s16(hz_ref, t16 + 1)

    t16_16 = t16.astype(jnp.int16)
    for i in range(n_ch):
        sl = slice(i * c_ch, (i + 1) * c_ch)
        m_eq = hz_ref[:, sl] == t16_16
        fi1 = jax.lax.bitcast_convert_type(f_ref[:, sl], jnp.int32)
        lo15 = (jax.lax.shift_right_logical(fi1, 1)
                & jnp.int32(0x7FFF)).astype(jnp.int16)
        hz_ref[:, sl] = jnp.where(m_eq, lo15, jnp.int16(-1))

    def body_lo(b, q):
        bit = jnp.int32(1) << (jnp.int32(14) - b)
        cand = q | bit
        cnt = c_gt + _count_ge_s16(hz_ref, cand)
        return jnp.where(cnt >= K, cand, q)

    q = jax.lax.fori_loop(0, 15, body_lo, jnp.zeros((tm, 1), jnp.int32))

    cand_pat = (t16 << 16) | (q << 1) | 1
    cand_f = jax.lax.bitcast_convert_type(cand_pat, jnp.float32)
    n_sl = 8
    c_sl = f_dim // n_sl
    parts = [
        jnp.sum(
            (f_ref[:, i * c_sl : (i + 1) * c_sl] >= cand_f).astype(jnp.float32),
            axis=1, keepdims=True)
        for i in range(n_sl)
    ]
    cnt_full = functools.reduce(jnp.add, parts).astype(jnp.int32)
    t_pat = jnp.where(cnt_full >= K, cand_pat, cand_pat ^ 1)
    t_ref[...] = jax.lax.bitcast_convert_type(t_pat, jnp.float32)


def _threshold(f, tm):
    n, f_dim = f.shape
    return pl.pallas_call(
        _threshold_body,
        grid=(n // tm,),
        in_specs=[pl.BlockSpec((tm, f_dim), lambda i: (i, 0))],
        out_specs=pl.BlockSpec((tm, 1), lambda i: (i, 0)),
        out_shape=jax.ShapeDtypeStruct((n, 1), jnp.float32),
        scratch_shapes=[pltpu.VMEM((tm, f_dim), jnp.int16)],
    )(f)


def _decode_body(f_ref, t_ref, dec_ref, bdec_ref, o_ref, vm_ref):
    j = pl.program_id(1)
    tf_ = t_ref[...]
    tf_dim = f_ref.shape[1]
    n_ch = 4
    c_ch = tf_dim // n_ch
    for i in range(n_ch):
        sl = slice(i * c_ch, (i + 1) * c_ch)
        fb = f_ref[:, sl]
        vm_ref[:, sl] = jnp.where(
            (fb >= tf_) & (fb > 0.0), fb, 0.0).astype(jnp.bfloat16)
    part = jax.lax.dot_general(
        vm_ref[...], dec_ref[...],
        dimension_numbers=(((1,), (0,)), ((), ())),
        preferred_element_type=jnp.float32,
    )

    @pl.when(j == 0)
    def _():
        o_ref[...] = part + bdec_ref[...]

    @pl.when(j > 0)
    def _():
        o_ref[...] += part


def _decode(f, t, decoder, b_dec, tm, tf):
    n, f_dim = f.shape
    d = decoder.shape[1]
    grid = (n // tm, f_dim // tf)
    return pl.pallas_call(
        _decode_body,
        grid=grid,
        in_specs=[
            pl.BlockSpec((tm, tf), lambda i, j: (i, j)),
            pl.BlockSpec((tm, 1), lambda i, j: (i, 0)),
            pl.BlockSpec((tf, d), lambda i, j: (j, 0)),
            pl.BlockSpec((1, d), lambda i, j: (0, 0)),
        ],
        out_specs=pl.BlockSpec((tm, d), lambda i, j: (i, 0)),
        out_shape=jax.ShapeDtypeStruct((n, d), jnp.float32),
        scratch_shapes=[pltpu.VMEM((tm, tf), jnp.bfloat16)],
    )(f, t, decoder, b_dec.reshape(1, d))


@functools.partial(jax.jit, static_argnames=())
def kernel(x, W_enc, b_enc, decoder, b_dec):
    orig_shape = x.shape
    x2 = x.reshape(-1, orig_shape[-1])
    n, d = x2.shape
    f_dim = W_enc.shape[0]

    tm_e = min(512, n)
    tn_e = min(1024, f_dim)
    f = _encode(x2, W_enc, b_enc, b_dec, tm_e, tn_e)

    tm_t = min(256, n)
    t = _threshold(f, tm_t)

    tm_d = min(512, n)
    tf_d = min(2048, f_dim)
    x_hat = _decode(f, t, decoder.astype(jnp.bfloat16), b_dec, tm_d, tf_d)
    return x_hat.reshape(orig_shape)
